# Initial kernel scaffold; baseline (speedup 1.0000x reference)
import functools

import jax
import jax.numpy as jnp
from jax import lax
from jax.experimental import pallas as pl
from jax.experimental.pallas import tpu as pltpu
from jax.experimental.pallas import tpu_sc as plsc

_B = 16384
_L = 200
_TOT = _B * _L
_ROW = 128
_TOT_ROWS = _TOT // _ROW
_NW = 32
_ROWS_W = _TOT_ROWS // _NW
_CR = 100
_NCHUNK = _ROWS_W // _CR


def _make_gather():
    info = plsc.get_sparse_core_info()
    nc = info.num_cores
    mesh = plsc.VectorSubcoreMesh(core_axis_name="c", subcore_axis_name="s")

    @functools.partial(
        pl.kernel,
        mesh=mesh,
        out_type=jax.ShapeDtypeStruct((_TOT_ROWS, _ROW), jnp.float32),
        scratch_types=[
            pltpu.VMEM((_CR, _ROW), jnp.int32),
            pltpu.VMEM((_CR, _ROW), jnp.float32),
            pltpu.SemaphoreType.DMA,
        ],
    )
    def gather_k(dist_hbm, idx_hbm, out_hbm, idx_v, out_v, sem):
        wid = lax.axis_index("s") * nc + lax.axis_index("c")
        row_base = wid * _ROWS_W

        def chunk(ci, carry):
            row0 = row_base + ci * _CR
            pltpu.sync_copy(idx_hbm.at[pl.ds(row0, _CR)], idx_v)

            def fire(r, c):
                pltpu.async_copy(dist_hbm.at[idx_v.at[r]], out_v.at[r], sem)
                return c

            lax.fori_loop(0, _CR, fire, 0)
            pltpu.make_async_copy(
                idx_hbm.at[pl.ds(row0, _CR)], out_v, sem
            ).wait()
            pltpu.sync_copy(out_v, out_hbm.at[pl.ds(row0, _CR)])
            return carry

        lax.fori_loop(0, _NCHUNK, chunk, 0)

    return gather_k


_gather = _make_gather()


def kernel(indices, distance):
    idx = indices.astype(jnp.int32).reshape(_TOT_ROWS, _ROW)
    out = _gather(distance, idx)
    return out.reshape(_B, _L)


# speedup vs baseline: 138.4562x; 138.4562x over previous
import functools

import jax
import jax.numpy as jnp
from jax import lax
from jax.experimental import pallas as pl
from jax.experimental.pallas import tpu as pltpu
from jax.experimental.pallas import tpu_sc as plsc

_B = 16384
_L = 200
_TOT = _B * _L
_ROW = 128
_TOT_ROWS = _TOT // _ROW
_NW = 32
_ROWS_W = _TOT_ROWS // _NW
_CR = 160
_NCHUNK = _ROWS_W // _CR


def _make_gather():
    info = plsc.get_sparse_core_info()
    nc = info.num_cores
    mesh = plsc.VectorSubcoreMesh(core_axis_name="c", subcore_axis_name="s")

    @functools.partial(
        pl.kernel,
        mesh=mesh,
        out_type=jax.ShapeDtypeStruct((_TOT_ROWS, _ROW), jnp.float32),
        scratch_types=[
            pltpu.VMEM((_CR, _ROW), jnp.int32),
            pltpu.VMEM((_CR, _ROW), jnp.float32),
            pltpu.SemaphoreType.DMA,
        ],
    )
    def gather_k(dist_hbm, idx_hbm, out_hbm, idx_v, out_v, sem):
        wid = lax.axis_index("s") * nc + lax.axis_index("c")
        row_base = wid * _ROWS_W

        def chunk(ci, carry):
            row0 = row_base + ci * _CR
            pltpu.sync_copy(idx_hbm.at[pl.ds(row0, _CR)], idx_v)

            def fire(r, c):
                pltpu.async_copy(dist_hbm.at[idx_v.at[r]], out_v.at[r], sem)
                return c

            lax.fori_loop(0, _CR, fire, 0)
            pltpu.make_async_copy(
                out_hbm.at[pl.ds(row0, _CR)], out_v, sem
            ).wait()
            pltpu.sync_copy(out_v, out_hbm.at[pl.ds(row0, _CR)])
            return carry

        lax.fori_loop(0, _NCHUNK, chunk, 0)

    return gather_k


_gather = _make_gather()


def kernel(indices, distance):
    idx = indices.astype(jnp.int32).reshape(_TOT_ROWS, _ROW)
    out = _gather(distance, idx)
    return out.reshape(_B, _L)
